# baseline (speedup 1.0000x reference)
import jax
import jax.numpy as jnp
from jax.experimental import pallas as pl
from jax.experimental.pallas import tpu as pltpu

T = 16384
D = 4096
E = 64
TILE_T = 512


def _router_kernel(x_ref, w_ref, ow_ref, oi_ref):
    logits_t = jax.lax.dot_general(
        w_ref[...], x_ref[...],
        dimension_numbers=(((1,), (1,)), ((), ())),
        preferred_element_type=jnp.float32,
    )

    val = logits_t
    ind = jax.lax.broadcasted_iota(jnp.int32, (E, TILE_T), 0)
    k = E
    while k > 1:
        k //= 2
        a, b = val[:k], val[k:]
        ia, ib = ind[:k], ind[k:]
        gt = b > a
        eq = b == a
        val = jnp.where(gt, b, a)
        ind = jnp.where(eq, jnp.minimum(ia, ib), jnp.where(gt, ib, ia))

    ex = jnp.exp(logits_t - val)
    k = E
    while k > 1:
        k //= 2
        ex = ex[:k] + ex[k:]
    ow_ref[...] = 1.0 / ex[0]
    oi_ref[...] = ind[0]


def kernel(x, W):
    grid = (T // TILE_T,)
    ow, oi = pl.pallas_call(
        _router_kernel,
        grid=grid,
        in_specs=[
            pl.BlockSpec((TILE_T, D), lambda i: (i, 0)),
            pl.BlockSpec((E, D), lambda i: (0, 0)),
        ],
        out_specs=[
            pl.BlockSpec((TILE_T,), lambda i: (i,)),
            pl.BlockSpec((TILE_T,), lambda i: (i,)),
        ],
        out_shape=[
            jax.ShapeDtypeStruct((T,), jnp.float32),
            jax.ShapeDtypeStruct((T,), jnp.int32),
        ],
        compiler_params=pltpu.CompilerParams(
            dimension_semantics=("parallel",),
        ),
    )(x, W)
    return (ow, oi)
